# Initial kernel scaffold; baseline (speedup 1.0000x reference)
import functools

import jax
import jax.numpy as jnp
from jax import lax
from jax.experimental import pallas as pl
from jax.experimental.pallas import tpu as pltpu
from jax.experimental.pallas import tpu_sc as plsc

B, S, D = 16, 2048, 1024
HALF = (S - 2) // 2


def _shift_body(bank_hbm, refresh_hbm, out_hbm, head_v, avg_v, sem):
    cid = lax.axis_index("c")
    sid = lax.axis_index("s")
    wid = sid * 2 + cid
    b = wid // 2
    h = wid % 2

    src_lo = 2 + h * HALF
    dst_lo = 1 + h * HALF
    copy = pltpu.make_async_copy(
        bank_hbm.at[b, pl.ds(src_lo, HALF)],
        out_hbm.at[b, pl.ds(dst_lo, HALF)],
        sem,
    )
    copy.start()

    @pl.when(h == 0)
    def _head():
        pltpu.sync_copy(bank_hbm.at[b, pl.ds(0, 2)], head_v)
        for i in range(D // 16):
            sl = pl.ds(i * 16, 16)
            avg_v[0, sl] = 0.5 * (head_v[0, sl] + head_v[1, sl])
        pltpu.sync_copy(avg_v, out_hbm.at[b, pl.ds(0, 1)])

    @pl.when(h == 1)
    def _tail():
        pltpu.sync_copy(refresh_hbm.at[b], out_hbm.at[b, pl.ds(S - 1, 1)])

    copy.wait()


@jax.jit
def _shift(bank_states, refresh_states):
    mesh = plsc.VectorSubcoreMesh(core_axis_name="c", subcore_axis_name="s")
    return pl.kernel(
        _shift_body,
        mesh=mesh,
        out_type=jax.ShapeDtypeStruct((B, S, D), jnp.float32),
        scratch_types=[
            pltpu.VMEM((2, D), jnp.float32),
            pltpu.VMEM((1, D), jnp.float32),
            pltpu.SemaphoreType.DMA,
        ],
    )(bank_states, refresh_states)


def kernel(bank_states, refresh_states):
    return _shift(bank_states, refresh_states)


# speedup vs baseline: 1.7809x; 1.7809x over previous
import functools

import jax
import jax.numpy as jnp
from jax import lax
from jax.experimental import pallas as pl
from jax.experimental.pallas import tpu as pltpu
from jax.experimental.pallas import tpu_sc as plsc

B, S, D = 16, 2048, 1024
L = 16
K = 32
NB = 2
HALF = S // 2
G = HALF // K


def _copy_row(dst_ref, dst_row, src_ref, src_row):
    for c in range(D // L):
        sl = pl.ds(c * L, L)
        dst_ref[dst_row, sl] = src_ref[src_row, sl]


def _body(bank, refresh, out, buf, bnd, sem_g, sem_b, sem_s):
    cid = lax.axis_index("c")
    sid = lax.axis_index("s")
    wid = sid * 2 + cid
    b = wid // 2
    h = wid % 2
    base = h * HALF

    @pl.when(h == 0)
    def _():
        pltpu.make_async_copy(bank.at[b, pl.ds(HALF, 1)], bnd, sem_b).start()

    @pl.when(h == 1)
    def _():
        pltpu.make_async_copy(refresh.at[b], bnd, sem_b).start()

    pltpu.make_async_copy(refresh.at[b], bnd, sem_b).wait()

    def gather(g, seg):
        cp = pltpu.make_async_copy(
            bank.at[b, pl.ds(base + g * K, K)],
            buf.at[pl.ds(seg * K, K)],
            sem_g,
        )
        cp.start()
        cp.wait()

    def scatter(g, seg):
        cp = pltpu.make_async_copy(
            buf.at[pl.ds(seg * K, K)],
            out.at[b, pl.ds(base + g * K, K)],
            sem_s,
        )
        cp.start()
        cp.wait()

    def compute(gm1, seg, seg_next, last):
        sb = seg * K
        is_avg = jnp.logical_and(gm1 == 0, h == 0)

        @pl.when(is_avg)
        def _():
            for c in range(D // L):
                sl = pl.ds(c * L, L)
                buf[sb, sl] = 0.5 * (buf[sb, sl] + buf[sb + 1, sl])

        i0 = jnp.where(is_avg, 1, 0)

        def row(i, carry):
            _copy_row(buf, sb + i, buf, sb + i + 1)
            return carry

        lax.fori_loop(i0, K - 1, row, 0)

        if last:
            _copy_row(buf, sb + K - 1, bnd, 0)
        else:
            _copy_row(buf, sb + K - 1, buf, seg_next * K)

    gather(0, 0)

    def chunk(g, carry):
        seg = g % NB
        gather(g, seg)
        gm1 = g - 1
        compute(gm1, gm1 % NB, seg, last=False)
        scatter(gm1, gm1 % NB)
        return carry

    lax.fori_loop(1, G, chunk, 0)

    last_seg = (G - 1) % NB
    compute(G - 1, last_seg, last_seg, last=True)
    scatter(G - 1, last_seg)


@jax.jit
def _shift(bank_states, refresh_states):
    mesh = plsc.VectorSubcoreMesh(core_axis_name="c", subcore_axis_name="s")
    return pl.kernel(
        _body,
        mesh=mesh,
        out_type=jax.ShapeDtypeStruct((B, S, D), jnp.float32),
        scratch_types=[
            pltpu.VMEM((NB * K, D), jnp.float32),
            pltpu.VMEM((1, D), jnp.float32),
            pltpu.SemaphoreType.DMA,
            pltpu.SemaphoreType.DMA,
            pltpu.SemaphoreType.DMA,
        ],
    )(bank_states, refresh_states)


def kernel(bank_states, refresh_states):
    return _shift(bank_states, refresh_states)


# speedup vs baseline: 2.5430x; 1.4280x over previous
import jax
import jax.numpy as jnp
from jax import lax
from jax.experimental import pallas as pl
from jax.experimental.pallas import tpu as pltpu
from jax.experimental.pallas import tpu_sc as plsc

B, S, D = 16, 2048, 1024
L = 16
K = 32
NB = 3
HALF = S // 2
G = HALF // K


def _copy_row(dst_ref, dst_row, src_ref, src_row):
    for c in range(D // L):
        sl = pl.ds(c * L, L)
        dst_ref[dst_row, sl] = src_ref[src_row, sl]


def _body(bank, refresh, out, buf, bnd, sem_g, sem_b, sem_s):
    cid = lax.axis_index("c")
    sid = lax.axis_index("s")
    wid = sid * 2 + cid
    b = wid // 2
    h = wid % 2
    base = h * HALF

    @pl.when(h == 0)
    def _():
        pltpu.make_async_copy(bank.at[b, pl.ds(HALF, 1)], bnd, sem_b).start()

    @pl.when(h == 1)
    def _():
        pltpu.make_async_copy(refresh.at[b], bnd, sem_b).start()

    def gather_start(g, seg):
        pltpu.make_async_copy(
            bank.at[b, pl.ds(base + g * K, K)],
            buf.at[pl.ds(seg * K, K)],
            sem_g,
        ).start()

    def gather_wait():
        pltpu.make_async_copy(
            bank.at[b, pl.ds(base, K)], buf.at[pl.ds(0, K)], sem_g
        ).wait()

    def scatter_start(g, seg):
        pltpu.make_async_copy(
            buf.at[pl.ds(seg * K, K)],
            out.at[b, pl.ds(base + g * K, K)],
            sem_s,
        ).start()

    def scatter_wait():
        pltpu.make_async_copy(
            buf.at[pl.ds(0, K)], out.at[b, pl.ds(base, K)], sem_s
        ).wait()

    gather_start(0, 0)
    gather_wait()
    pltpu.make_async_copy(refresh.at[b], bnd, sem_b).wait()

    def chunk(g, carry):
        seg = g % NB
        segn = (g + 1) % NB
        sb = seg * K

        @pl.when(g >= 2)
        def _():
            scatter_wait()

        @pl.when(g + 1 < G)
        def _():
            gather_start(g + 1, segn)

        is_avg = jnp.logical_and(g == 0, h == 0)

        @pl.when(is_avg)
        def _():
            for c in range(D // L):
                sl = pl.ds(c * L, L)
                buf[sb, sl] = 0.5 * (buf[sb, sl] + buf[sb + 1, sl])

        i0 = jnp.where(is_avg, 1, 0)

        def row(i, c2):
            _copy_row(buf, sb + i, buf, sb + i + 1)
            return c2

        lax.fori_loop(i0, K - 1, row, 0)

        @pl.when(g + 1 < G)
        def _():
            gather_wait()
            _copy_row(buf, sb + K - 1, buf, segn * K)

        @pl.when(g + 1 == G)
        def _():
            _copy_row(buf, sb + K - 1, bnd, 0)

        scatter_start(g, seg)
        return carry

    lax.fori_loop(0, G, chunk, 0)
    scatter_wait()
    scatter_wait()


@jax.jit
def _shift(bank_states, refresh_states):
    mesh = plsc.VectorSubcoreMesh(core_axis_name="c", subcore_axis_name="s")
    return pl.kernel(
        _body,
        mesh=mesh,
        out_type=jax.ShapeDtypeStruct((B, S, D), jnp.float32),
        scratch_types=[
            pltpu.VMEM((NB * K, D), jnp.float32),
            pltpu.VMEM((1, D), jnp.float32),
            pltpu.SemaphoreType.DMA,
            pltpu.SemaphoreType.DMA,
            pltpu.SemaphoreType.DMA,
        ],
    )(bank_states, refresh_states)


def kernel(bank_states, refresh_states):
    return _shift(bank_states, refresh_states)
